# baseline (speedup 1.0000x reference)
import functools

import jax
import jax.numpy as jnp
from jax import lax
from jax.experimental import pallas as pl
from jax.experimental.pallas import tpu as pltpu
from jax.experimental.pallas import tpu_sc as plsc

N_NODES = 10000
NUM_EDGES = 320000
D_IN = 128
D_H = 128
D_OUT = 64
ROW_BLK = 2000

SC_CORES = 2
SC_TILES = 16
NW = SC_CORES * SC_TILES
CH = 128
EPW = 10240
NCH = EPW // CH
E_PAD = NW * EPW
N_PAD = 10240
RPT = N_PAD // SC_TILES


def _pre_body(xs_ref, xd_ref, ws_ref, wd_ref, avs_ref, avd_ref,
              hs0_ref, hs1_ref, asrc_ref, adst_ref):
    hs = jnp.dot(xs_ref[...], ws_ref[...], preferred_element_type=jnp.float32)
    hs0_ref[...] = hs[:, :D_H // 2]
    hs1_ref[...] = hs[:, D_H // 2:]
    asrc_ref[...] = jnp.sum(hs * avs_ref[...][None, :], axis=1, keepdims=True)
    wda = jnp.dot(wd_ref[...], avd_ref[...][:, None],
                  preferred_element_type=jnp.float32)
    adst_ref[...] = jnp.dot(xd_ref[...], wda, preferred_element_type=jnp.float32)


def _gat_pre(x_src, x_dst, p):
    n = x_src.shape[0]
    grid = n // ROW_BLK
    return pl.pallas_call(
        _pre_body,
        grid=(grid,),
        in_specs=[
            pl.BlockSpec((ROW_BLK, x_src.shape[1]), lambda m: (m, 0)),
            pl.BlockSpec((ROW_BLK, x_dst.shape[1]), lambda m: (m, 0)),
            pl.BlockSpec(p["W_src"].shape, lambda m: (0, 0)),
            pl.BlockSpec(p["W_dst"].shape, lambda m: (0, 0)),
            pl.BlockSpec(p["att_src"].shape, lambda m: (0,)),
            pl.BlockSpec(p["att_dst"].shape, lambda m: (0,)),
        ],
        out_specs=[
            pl.BlockSpec((ROW_BLK, D_H // 2), lambda m: (m, 0)),
            pl.BlockSpec((ROW_BLK, D_H // 2), lambda m: (m, 0)),
            pl.BlockSpec((ROW_BLK, 1), lambda m: (m, 0)),
            pl.BlockSpec((ROW_BLK, 1), lambda m: (m, 0)),
        ],
        out_shape=[
            jax.ShapeDtypeStruct((n, D_H // 2), jnp.float32),
            jax.ShapeDtypeStruct((n, D_H // 2), jnp.float32),
            jax.ShapeDtypeStruct((n, 1), jnp.float32),
            jax.ShapeDtypeStruct((n, 1), jnp.float32),
        ],
    )(x_src, x_dst, p["W_src"], p["W_dst"], p["att_src"], p["att_dst"])


def _post_body(acc0_ref, acc1_ref, den_ref, bias_ref, out_ref):
    acc0 = acc0_ref[...]
    acc1 = acc1_ref[...]
    den = den_ref[...]
    acc_t = jnp.concatenate([acc0[0] + acc0[1], acc1[0] + acc1[1]], axis=-1)
    den_t = den[0] + den[1]
    out = acc_t / jnp.maximum(den_t, 1e-16)
    out_ref[...] = jnp.maximum(out + bias_ref[...][None, :], 0.0)


def _gat_post(acc0, acc1, den, bias, n):
    grid = n // ROW_BLK
    return pl.pallas_call(
        _post_body,
        grid=(grid,),
        in_specs=[
            pl.BlockSpec((SC_CORES, ROW_BLK, D_H // 2), lambda m: (0, m, 0)),
            pl.BlockSpec((SC_CORES, ROW_BLK, D_H // 2), lambda m: (0, m, 0)),
            pl.BlockSpec((SC_CORES, ROW_BLK, 1), lambda m: (0, m, 0)),
            pl.BlockSpec(bias.shape, lambda m: (0,)),
        ],
        out_specs=pl.BlockSpec((ROW_BLK, D_H), lambda m: (m, 0)),
        out_shape=jax.ShapeDtypeStruct((n, D_H), jnp.float32),
    )(acc0, acc1, den, bias)


def _final_body(x_ref, w1_ref, b1_ref, w2_ref, b2_ref, out_ref):
    h = jnp.dot(x_ref[...], w1_ref[...], preferred_element_type=jnp.float32)
    h = jnp.maximum(h + b1_ref[...][None, :], 0.0)
    y = jnp.dot(h, w2_ref[...], preferred_element_type=jnp.float32)
    out_ref[...] = y + b2_ref[...][None, :]


def _final_mlp(x, w1, b1, w2, b2):
    n = x.shape[0]
    grid = n // ROW_BLK
    return pl.pallas_call(
        _final_body,
        grid=(grid,),
        in_specs=[
            pl.BlockSpec((ROW_BLK, D_H), lambda m: (m, 0)),
            pl.BlockSpec(w1.shape, lambda m: (0, 0)),
            pl.BlockSpec(b1.shape, lambda m: (0,)),
            pl.BlockSpec(w2.shape, lambda m: (0, 0)),
            pl.BlockSpec(b2.shape, lambda m: (0,)),
        ],
        out_specs=pl.BlockSpec((ROW_BLK, 1), lambda m: (m, 0)),
        out_shape=jax.ShapeDtypeStruct((n, 1), jnp.float32),
    )(x, w1, b1, w2, b2)


D_HALF = D_H // 2


SCALE_UNROLL = 4


def _sc_conv_body(src_hbm, dst_hbm, asrc_hbm, adst_hbm, hs0_hbm, hs1_hbm,
                  acc0_out, acc1_out, den_out,
                  src_v, dst_v, asrc_v, adst_v, rows0_v, rows1_v,
                  exb0_v, exb1_v, exall_v,
                  acc_sh, den_sh, sem_g, sem_s):
    c = lax.axis_index("c")
    s = lax.axis_index("s")
    wid = s * SC_CORES + c
    rows = (rows0_v, rows1_v)
    exb = (exb0_v, exb1_v)
    pltpu.sync_copy(src_hbm.at[wid], src_v)
    pltpu.sync_copy(dst_hbm.at[wid], dst_v)
    pltpu.sync_copy(asrc_hbm, asrc_v)
    pltpu.sync_copy(adst_hbm, adst_v)

    def _zero_rows():
        def _zrow(r, carry):
            for v in range(D_HALF // 16):
                rows0_v[r, pl.ds(v * 16, 16)] = jnp.zeros((16,), jnp.float32)
            return carry
        lax.fori_loop(0, CH, _zrow, 0)

    def _zero_acc(include_den):
        _zero_rows()
        if include_den:
            for i in range(CH // 16):
                exb0_v[pl.ds(i * 16, 16)] = jnp.zeros((16,), jnp.float32)
        for k in range(RPT // CH):
            pltpu.sync_copy(rows0_v, acc_sh.at[pl.ds(row0 + k * CH, CH)])
            if include_den:
                pltpu.sync_copy(exb0_v, den_sh.at[pl.ds(row0 + k * CH, CH)])

    def _scale_rows(rv, ev):
        def _scale(g, carry2):
            for u in range(SCALE_UNROLL):
                r = g * SCALE_UNROLL + u
                exr = plsc.load_gather(ev, [jnp.full((16,), r, jnp.int32)])
                for v in range(D_HALF // 16):
                    rv[r, pl.ds(v * 16, 16)] = rv[r, pl.ds(v * 16, 16)] * exr
            return carry2
        lax.fori_loop(0, CH // SCALE_UNROLL, _scale, 0)

    row0 = s * RPT

    def _run_pass(hs_hbm, first_pass):
        pltpu.async_copy(hs_hbm.at[src_v.at[0]], rows[0], sem_g)

        def _slot(j, p, drain_prev):
            q = 1 - p
            if first_pass:
                base = wid * EPW + j * CH
                for i in range(CH // 16):
                    sv = src_v[j, pl.ds(i * 16, 16)]
                    dv = dst_v[j, pl.ds(i * 16, 16)]
                    a = (plsc.load_gather(asrc_v, [sv])
                         + plsc.load_gather(adst_v, [dv]))
                    a = jnp.maximum(a, 0.2 * a)
                    ex = jnp.exp(a)
                    eid = base + i * 16 + lax.iota(jnp.int32, 16)
                    ex = jnp.where(eid < NUM_EDGES, ex, 0.0)
                    exb[p][pl.ds(i * 16, 16)] = ex
                    exall_v[j, pl.ds(i * 16, 16)] = ex
            else:
                for i in range(CH // 16):
                    exb[p][pl.ds(i * 16, 16)] = exall_v[j, pl.ds(i * 16, 16)]
            pltpu.make_async_copy(hs_hbm.at[src_v.at[j]], rows[p],
                                  sem_g).wait()
            _scale_rows(rows[p], exb[p])
            if drain_prev:
                pltpu.make_async_copy(rows[q], acc_sh.at[dst_v.at[j - 1]],
                                      sem_s).wait()
                if first_pass:
                    pltpu.make_async_copy(exb[q], den_sh.at[dst_v.at[j - 1]],
                                          sem_s).wait()
            gj = jnp.minimum(j + 1, NCH - 1)
            pltpu.async_copy(hs_hbm.at[src_v.at[gj]], rows[q], sem_g)
            pltpu.async_copy(rows[p], acc_sh.at[dst_v.at[j]], sem_s, add=True)
            if first_pass:
                pltpu.async_copy(exb[p], den_sh.at[dst_v.at[j]], sem_s,
                                 add=True)

        _slot(0, 0, False)
        _slot(1, 1, True)

        def _pair(t, carry):
            _slot(2 * t, 0, True)
            _slot(2 * t + 1, 1, True)
            return carry
        lax.fori_loop(1, NCH // 2, _pair, 0)

        pltpu.make_async_copy(rows[1], acc_sh.at[dst_v.at[NCH - 1]],
                              sem_s).wait()
        if first_pass:
            pltpu.make_async_copy(exb[1], den_sh.at[dst_v.at[NCH - 1]],
                                  sem_s).wait()
        pltpu.make_async_copy(hs_hbm.at[src_v.at[NCH - 1]], rows[0],
                              sem_g).wait()

    _zero_acc(True)
    plsc.subcore_barrier()
    _run_pass(hs0_hbm, True)
    plsc.subcore_barrier()

    for k in range(RPT // CH):
        r0 = row0 + k * CH
        pltpu.sync_copy(acc_sh.at[pl.ds(r0, CH)], rows0_v)
        pltpu.sync_copy(rows0_v, acc0_out.at[c, pl.ds(r0, CH)])
        pltpu.sync_copy(den_sh.at[pl.ds(r0, CH)], exb0_v)
        pltpu.sync_copy(exb0_v, den_out.at[c, pl.ds(r0, CH)])
    _zero_acc(False)
    plsc.subcore_barrier()

    _run_pass(hs1_hbm, False)
    plsc.subcore_barrier()

    for k in range(RPT // CH):
        r0 = row0 + k * CH
        pltpu.sync_copy(acc_sh.at[pl.ds(r0, CH)], rows0_v)
        pltpu.sync_copy(rows0_v, acc1_out.at[c, pl.ds(r0, CH)])


def _sc_conv(src_r, dst_r, a_src, a_dst, hs0, hs1):
    mesh = plsc.VectorSubcoreMesh(core_axis_name="c", subcore_axis_name="s")
    f = pl.kernel(
        _sc_conv_body,
        out_type=[
            jax.ShapeDtypeStruct((SC_CORES, N_PAD, D_HALF), jnp.float32),
            jax.ShapeDtypeStruct((SC_CORES, N_PAD, D_HALF), jnp.float32),
            jax.ShapeDtypeStruct((SC_CORES, N_PAD), jnp.float32),
        ],
        mesh=mesh,
        scratch_types=[
            pltpu.VMEM((NCH, CH), jnp.int32),
            pltpu.VMEM((NCH, CH), jnp.int32),
            pltpu.VMEM((N_NODES,), jnp.float32),
            pltpu.VMEM((N_NODES,), jnp.float32),
            pltpu.VMEM((CH, D_HALF), jnp.float32),
            pltpu.VMEM((CH, D_HALF), jnp.float32),
            pltpu.VMEM((CH,), jnp.float32),
            pltpu.VMEM((CH,), jnp.float32),
            pltpu.VMEM((NCH, CH), jnp.float32),
            pltpu.VMEM_SHARED((N_PAD, D_HALF), jnp.float32),
            pltpu.VMEM_SHARED((N_PAD,), jnp.float32),
            pltpu.SemaphoreType.DMA,
            pltpu.SemaphoreType.DMA,
        ],
        compiler_params=pltpu.CompilerParams(needs_layout_passes=False,
                                             use_tc_tiling_on_sc=False),
    )
    return f(src_r, dst_r, a_src, a_dst, hs0, hs1)


def _sparse_middle(hs0, hs1, a_src, a_dst, src_r, dst_r):
    acc0, acc1, den = _sc_conv(src_r, dst_r, a_src.reshape(-1),
                               a_dst.reshape(-1), hs0, hs1)
    return acc0, acc1, den.reshape(SC_CORES, N_PAD, 1)


def _prep_edges(ei):
    ei = ei.astype(jnp.int32)
    src = jnp.pad(ei[0], (0, E_PAD - NUM_EDGES))
    dst = jnp.pad(ei[1], (0, E_PAD - NUM_EDGES))
    return src.reshape(NW, NCH, CH), dst.reshape(NW, NCH, CH)


def kernel(x_inst, x_net, edge_index_i2n, edge_index_n2i, params):
    edges = {
        "i2n": _prep_edges(edge_index_i2n),
        "n2i": _prep_edges(edge_index_n2i),
    }
    for l in range(2):
        p = params["conv"][l]
        outs = []
        for rel, x_src, x_dst in (
            ("i2n", x_inst, x_net),
            ("n2i", x_net, x_inst),
        ):
            pp = p[rel]
            src_r, dst_r = edges[rel]
            hs0, hs1, a_src, a_dst = _gat_pre(x_src, x_dst, pp)
            acc0, acc1, den = _sparse_middle(hs0, hs1, a_src, a_dst,
                                             src_r, dst_r)
            outs.append(_gat_post(acc0, acc1, den, pp["bias"], x_dst.shape[0]))
        x_net, x_inst = outs
    x = _final_mlp(x_net, params["lin1_W"], params["lin1_b"],
                   params["lin2_W"], params["lin2_b"])
    return x


# speedup vs baseline: 1.2193x; 1.2193x over previous
import functools

import jax
import jax.numpy as jnp
from jax import lax
from jax.experimental import pallas as pl
from jax.experimental.pallas import tpu as pltpu
from jax.experimental.pallas import tpu_sc as plsc

N_NODES = 10000
NUM_EDGES = 320000
D_IN = 128
D_H = 128
D_OUT = 64
ROW_BLK = 2000

SC_CORES = 2
SC_TILES = 16
NW = SC_CORES * SC_TILES
CH = 128
EPW = 10240
NCH = EPW // CH
E_PAD = NW * EPW
N_PAD = 10240
RPT = N_PAD // SC_TILES


def _pre_body(xs_ref, xd_ref, ws_ref, wd_ref, avs_ref, avd_ref,
              hs0_ref, hs1_ref, asrc_ref, adst_ref):
    hs = jnp.dot(xs_ref[...], ws_ref[...], preferred_element_type=jnp.float32)
    hs0_ref[...] = hs[:, :D_H // 2]
    hs1_ref[...] = hs[:, D_H // 2:]
    asrc_ref[...] = jnp.sum(hs * avs_ref[...][None, :], axis=1, keepdims=True)
    wda = jnp.dot(wd_ref[...], avd_ref[...][:, None],
                  preferred_element_type=jnp.float32)
    adst_ref[...] = jnp.dot(xd_ref[...], wda, preferred_element_type=jnp.float32)


def _gat_pre(x_src, x_dst, p):
    n = x_src.shape[0]
    grid = n // ROW_BLK
    return pl.pallas_call(
        _pre_body,
        grid=(grid,),
        in_specs=[
            pl.BlockSpec((ROW_BLK, x_src.shape[1]), lambda m: (m, 0)),
            pl.BlockSpec((ROW_BLK, x_dst.shape[1]), lambda m: (m, 0)),
            pl.BlockSpec(p["W_src"].shape, lambda m: (0, 0)),
            pl.BlockSpec(p["W_dst"].shape, lambda m: (0, 0)),
            pl.BlockSpec(p["att_src"].shape, lambda m: (0,)),
            pl.BlockSpec(p["att_dst"].shape, lambda m: (0,)),
        ],
        out_specs=[
            pl.BlockSpec((ROW_BLK, D_H // 2), lambda m: (m, 0)),
            pl.BlockSpec((ROW_BLK, D_H // 2), lambda m: (m, 0)),
            pl.BlockSpec((ROW_BLK, 1), lambda m: (m, 0)),
            pl.BlockSpec((ROW_BLK, 1), lambda m: (m, 0)),
        ],
        out_shape=[
            jax.ShapeDtypeStruct((n, D_H // 2), jnp.float32),
            jax.ShapeDtypeStruct((n, D_H // 2), jnp.float32),
            jax.ShapeDtypeStruct((n, 1), jnp.float32),
            jax.ShapeDtypeStruct((n, 1), jnp.float32),
        ],
    )(x_src, x_dst, p["W_src"], p["W_dst"], p["att_src"], p["att_dst"])


def _post_body(acc0_ref, acc1_ref, den_ref, bias_ref, out_ref):
    acc0 = acc0_ref[...]
    acc1 = acc1_ref[...]
    den = den_ref[...]
    acc_t = jnp.concatenate([acc0[0] + acc0[1], acc1[0] + acc1[1]], axis=-1)
    den_t = den[0] + den[1]
    out = acc_t / jnp.maximum(den_t, 1e-16)
    out_ref[...] = jnp.maximum(out + bias_ref[...][None, :], 0.0)


def _gat_post(acc0, acc1, den, bias, n):
    grid = n // ROW_BLK
    return pl.pallas_call(
        _post_body,
        grid=(grid,),
        in_specs=[
            pl.BlockSpec((SC_CORES, ROW_BLK, D_H // 2), lambda m: (0, m, 0)),
            pl.BlockSpec((SC_CORES, ROW_BLK, D_H // 2), lambda m: (0, m, 0)),
            pl.BlockSpec((SC_CORES, ROW_BLK, 1), lambda m: (0, m, 0)),
            pl.BlockSpec(bias.shape, lambda m: (0,)),
        ],
        out_specs=pl.BlockSpec((ROW_BLK, D_H), lambda m: (m, 0)),
        out_shape=jax.ShapeDtypeStruct((n, D_H), jnp.float32),
    )(acc0, acc1, den, bias)


def _final_body(x_ref, w1_ref, b1_ref, w2_ref, b2_ref, out_ref):
    h = jnp.dot(x_ref[...], w1_ref[...], preferred_element_type=jnp.float32)
    h = jnp.maximum(h + b1_ref[...][None, :], 0.0)
    y = jnp.dot(h, w2_ref[...], preferred_element_type=jnp.float32)
    out_ref[...] = y + b2_ref[...][None, :]


def _final_mlp(x, w1, b1, w2, b2):
    n = x.shape[0]
    grid = n // ROW_BLK
    return pl.pallas_call(
        _final_body,
        grid=(grid,),
        in_specs=[
            pl.BlockSpec((ROW_BLK, D_H), lambda m: (m, 0)),
            pl.BlockSpec(w1.shape, lambda m: (0, 0)),
            pl.BlockSpec(b1.shape, lambda m: (0,)),
            pl.BlockSpec(w2.shape, lambda m: (0, 0)),
            pl.BlockSpec(b2.shape, lambda m: (0,)),
        ],
        out_specs=pl.BlockSpec((ROW_BLK, 1), lambda m: (m, 0)),
        out_shape=jax.ShapeDtypeStruct((n, 1), jnp.float32),
    )(x, w1, b1, w2, b2)


D_HALF = D_H // 2


SCALE_UNROLL = 4


def _sc_conv_body(src_hbm, dst_hbm, asrc_hbm, adst_hbm, hs0_hbm, hs1_hbm,
                  acc0_out, acc1_out, den_out,
                  src_v, dst_v, asrc_v, adst_v, rows0_v, rows1_v,
                  exb0_v, exb1_v, exall_v,
                  acc_sh, den_sh, sem_g, sem_s):
    c = lax.axis_index("c")
    s = lax.axis_index("s")
    wid = s * SC_CORES + c
    rows = (rows0_v, rows1_v)
    exb = (exb0_v, exb1_v)
    pltpu.sync_copy(src_hbm.at[wid], src_v)
    pltpu.sync_copy(dst_hbm.at[wid], dst_v)
    pltpu.sync_copy(asrc_hbm, asrc_v)
    pltpu.sync_copy(adst_hbm, adst_v)

    def _zero_rows():
        def _zrow(r, carry):
            for v in range(D_HALF // 16):
                rows0_v[r, pl.ds(v * 16, 16)] = jnp.zeros((16,), jnp.float32)
            return carry
        lax.fori_loop(0, CH, _zrow, 0)

    def _zero_acc(include_den):
        _zero_rows()
        if include_den:
            for i in range(CH // 16):
                exb0_v[pl.ds(i * 16, 16)] = jnp.zeros((16,), jnp.float32)
        for k in range(RPT // CH):
            pltpu.sync_copy(rows0_v, acc_sh.at[pl.ds(row0 + k * CH, CH)])
            if include_den:
                pltpu.sync_copy(exb0_v, den_sh.at[pl.ds(row0 + k * CH, CH)])

    def _scale_rows(rv, ev):
        @plsc.parallel_loop(0, CH, 1, unroll=SCALE_UNROLL)
        def _scale(r):
            exr = plsc.load_gather(ev, [jnp.full((16,), r, jnp.int32)])
            for v in range(D_HALF // 16):
                rv[r, pl.ds(v * 16, 16)] = rv[r, pl.ds(v * 16, 16)] * exr

    row0 = s * RPT

    def _run_pass(hs_hbm, first_pass):
        pltpu.async_copy(hs_hbm.at[src_v.at[0]], rows[0], sem_g)

        def _slot(j, p, drain_prev):
            q = 1 - p
            if first_pass:
                base = wid * EPW + j * CH
                for i in range(CH // 16):
                    sv = src_v[j, pl.ds(i * 16, 16)]
                    dv = dst_v[j, pl.ds(i * 16, 16)]
                    a = (plsc.load_gather(asrc_v, [sv])
                         + plsc.load_gather(adst_v, [dv]))
                    a = jnp.maximum(a, 0.2 * a)
                    ex = jnp.exp(a)
                    eid = base + i * 16 + lax.iota(jnp.int32, 16)
                    ex = jnp.where(eid < NUM_EDGES, ex, 0.0)
                    exb[p][pl.ds(i * 16, 16)] = ex
                    exall_v[j, pl.ds(i * 16, 16)] = ex
            else:
                for i in range(CH // 16):
                    exb[p][pl.ds(i * 16, 16)] = exall_v[j, pl.ds(i * 16, 16)]
            pltpu.make_async_copy(hs_hbm.at[src_v.at[j]], rows[p],
                                  sem_g).wait()
            gj = jnp.minimum(j + 1, NCH - 1)
            pltpu.async_copy(hs_hbm.at[src_v.at[gj]], rows[q], sem_g)

            _scale_rows(rows[p], exb[p])
            pltpu.sync_copy(rows[p], acc_sh.at[dst_v.at[j]], add=True)
            if first_pass:
                pltpu.sync_copy(exb[p], den_sh.at[dst_v.at[j]], add=True)

        def _pair(t, carry):
            _slot(2 * t, 0, True)
            _slot(2 * t + 1, 1, True)
            return carry
        lax.fori_loop(0, NCH // 2, _pair, 0)

        pltpu.make_async_copy(hs_hbm.at[src_v.at[NCH - 1]], rows[0],
                              sem_g).wait()

    _zero_acc(True)
    plsc.subcore_barrier()
    _run_pass(hs0_hbm, True)
    plsc.subcore_barrier()

    for k in range(RPT // CH):
        r0 = row0 + k * CH
        pltpu.sync_copy(acc_sh.at[pl.ds(r0, CH)], rows0_v)
        pltpu.sync_copy(rows0_v, acc0_out.at[c, pl.ds(r0, CH)])
        pltpu.sync_copy(den_sh.at[pl.ds(r0, CH)], exb0_v)
        pltpu.sync_copy(exb0_v, den_out.at[c, pl.ds(r0, CH)])
    _zero_acc(False)
    plsc.subcore_barrier()

    _run_pass(hs1_hbm, False)
    plsc.subcore_barrier()

    for k in range(RPT // CH):
        r0 = row0 + k * CH
        pltpu.sync_copy(acc_sh.at[pl.ds(r0, CH)], rows0_v)
        pltpu.sync_copy(rows0_v, acc1_out.at[c, pl.ds(r0, CH)])


def _sc_conv(src_r, dst_r, a_src, a_dst, hs0, hs1):
    mesh = plsc.VectorSubcoreMesh(core_axis_name="c", subcore_axis_name="s")
    f = pl.kernel(
        _sc_conv_body,
        out_type=[
            jax.ShapeDtypeStruct((SC_CORES, N_PAD, D_HALF), jnp.float32),
            jax.ShapeDtypeStruct((SC_CORES, N_PAD, D_HALF), jnp.float32),
            jax.ShapeDtypeStruct((SC_CORES, N_PAD), jnp.float32),
        ],
        mesh=mesh,
        scratch_types=[
            pltpu.VMEM((NCH, CH), jnp.int32),
            pltpu.VMEM((NCH, CH), jnp.int32),
            pltpu.VMEM((N_NODES,), jnp.float32),
            pltpu.VMEM((N_NODES,), jnp.float32),
            pltpu.VMEM((CH, D_HALF), jnp.float32),
            pltpu.VMEM((CH, D_HALF), jnp.float32),
            pltpu.VMEM((CH,), jnp.float32),
            pltpu.VMEM((CH,), jnp.float32),
            pltpu.VMEM((NCH, CH), jnp.float32),
            pltpu.VMEM_SHARED((N_PAD, D_HALF), jnp.float32),
            pltpu.VMEM_SHARED((N_PAD,), jnp.float32),
            pltpu.SemaphoreType.DMA,
            pltpu.SemaphoreType.DMA,
        ],
        compiler_params=pltpu.CompilerParams(needs_layout_passes=False,
                                             use_tc_tiling_on_sc=False),
    )
    return f(src_r, dst_r, a_src, a_dst, hs0, hs1)


def _sparse_middle(hs0, hs1, a_src, a_dst, src_r, dst_r):
    acc0, acc1, den = _sc_conv(src_r, dst_r, a_src.reshape(-1),
                               a_dst.reshape(-1), hs0, hs1)
    return acc0, acc1, den.reshape(SC_CORES, N_PAD, 1)


def _prep_edges(ei):
    ei = ei.astype(jnp.int32)
    src = jnp.pad(ei[0], (0, E_PAD - NUM_EDGES))
    dst = jnp.pad(ei[1], (0, E_PAD - NUM_EDGES))
    return src.reshape(NW, NCH, CH), dst.reshape(NW, NCH, CH)


def kernel(x_inst, x_net, edge_index_i2n, edge_index_n2i, params):
    edges = {
        "i2n": _prep_edges(edge_index_i2n),
        "n2i": _prep_edges(edge_index_n2i),
    }
    for l in range(2):
        p = params["conv"][l]
        outs = []
        for rel, x_src, x_dst in (
            ("i2n", x_inst, x_net),
            ("n2i", x_net, x_inst),
        ):
            pp = p[rel]
            src_r, dst_r = edges[rel]
            hs0, hs1, a_src, a_dst = _gat_pre(x_src, x_dst, pp)
            acc0, acc1, den = _sparse_middle(hs0, hs1, a_src, a_dst,
                                             src_r, dst_r)
            outs.append(_gat_post(acc0, acc1, den, pp["bias"], x_dst.shape[0]))
        x_net, x_inst = outs
    x = _final_mlp(x_net, params["lin1_W"], params["lin1_b"],
                   params["lin2_W"], params["lin2_b"])
    return x


# speedup vs baseline: 1.9118x; 1.5679x over previous
import functools

import jax
import jax.numpy as jnp
import numpy as np
from jax import lax
from jax.experimental import pallas as pl
from jax.experimental.pallas import tpu as pltpu
from jax.experimental.pallas import tpu_sc as plsc

_PERM32 = np.empty(32, np.int32)
_PERM32[0::2] = np.arange(16)
_PERM32[1::2] = np.arange(16, 32)


def _bf16_col_perm(width):
    return np.concatenate([g * 32 + _PERM32 for g in range(width // 32)])

N_NODES = 10000
NUM_EDGES = 320000
D_IN = 128
D_H = 128
D_OUT = 64
ROW_BLK = 2000

SC_CORES = 2
SC_TILES = 16
NW = SC_CORES * SC_TILES
CH = 128
EPW = 10240
NCH = EPW // CH
E_PAD = NW * EPW
N_PAD = 10240
RPT = N_PAD // SC_TILES


def _pre_body(xs_ref, xd_ref, w0_ref, w1_ref, ws_ref, wd_ref,
              avs_ref, avd_ref, hs0_ref, hs1_ref, asrc_ref, adst_ref):
    xs = xs_ref[...]
    hs0_ref[...] = jnp.dot(xs, w0_ref[...],
                           preferred_element_type=jnp.float32
                           ).astype(jnp.bfloat16)
    hs1_ref[...] = jnp.dot(xs, w1_ref[...],
                           preferred_element_type=jnp.float32
                           ).astype(jnp.bfloat16)
    wsa = jnp.dot(ws_ref[...], avs_ref[...][:, None],
                  preferred_element_type=jnp.float32)
    asrc_ref[...] = jnp.dot(xs, wsa, preferred_element_type=jnp.float32)
    wda = jnp.dot(wd_ref[...], avd_ref[...][:, None],
                  preferred_element_type=jnp.float32)
    adst_ref[...] = jnp.dot(xd_ref[...], wda, preferred_element_type=jnp.float32)


def _gat_pre(x_src, x_dst, p):
    n = x_src.shape[0]
    grid = n // ROW_BLK
    perm = _bf16_col_perm(D_H // 2)
    w0 = p["W_src"][:, :D_H // 2][:, perm]
    w1 = p["W_src"][:, D_H // 2:][:, perm]
    return pl.pallas_call(
        _pre_body,
        grid=(grid,),
        in_specs=[
            pl.BlockSpec((ROW_BLK, x_src.shape[1]), lambda m: (m, 0)),
            pl.BlockSpec((ROW_BLK, x_dst.shape[1]), lambda m: (m, 0)),
            pl.BlockSpec(w0.shape, lambda m: (0, 0)),
            pl.BlockSpec(w1.shape, lambda m: (0, 0)),
            pl.BlockSpec(p["W_src"].shape, lambda m: (0, 0)),
            pl.BlockSpec(p["W_dst"].shape, lambda m: (0, 0)),
            pl.BlockSpec(p["att_src"].shape, lambda m: (0,)),
            pl.BlockSpec(p["att_dst"].shape, lambda m: (0,)),
        ],
        out_specs=[
            pl.BlockSpec((ROW_BLK, D_H // 2), lambda m: (m, 0)),
            pl.BlockSpec((ROW_BLK, D_H // 2), lambda m: (m, 0)),
            pl.BlockSpec((ROW_BLK, 1), lambda m: (m, 0)),
            pl.BlockSpec((ROW_BLK, 1), lambda m: (m, 0)),
        ],
        out_shape=[
            jax.ShapeDtypeStruct((n, D_H // 2), jnp.bfloat16),
            jax.ShapeDtypeStruct((n, D_H // 2), jnp.bfloat16),
            jax.ShapeDtypeStruct((n, 1), jnp.float32),
            jax.ShapeDtypeStruct((n, 1), jnp.float32),
        ],
    )(x_src, x_dst, w0, w1, p["W_src"], p["W_dst"],
      p["att_src"], p["att_dst"])


def _post_body(acc0_ref, acc1_ref, den_ref, bias_ref, out_ref):
    acc0 = acc0_ref[...]
    acc1 = acc1_ref[...]
    den = den_ref[...]
    acc_t = jnp.concatenate([acc0[0] + acc0[1], acc1[0] + acc1[1]], axis=-1)
    den_t = den[0] + den[1]
    out = acc_t / jnp.maximum(den_t, 1e-16)
    out_ref[...] = jnp.maximum(out + bias_ref[...][None, :], 0.0)


def _gat_post(acc0, acc1, den, bias, n):
    grid = n // ROW_BLK
    return pl.pallas_call(
        _post_body,
        grid=(grid,),
        in_specs=[
            pl.BlockSpec((SC_CORES, ROW_BLK, D_H // 2), lambda m: (0, m, 0)),
            pl.BlockSpec((SC_CORES, ROW_BLK, D_H // 2), lambda m: (0, m, 0)),
            pl.BlockSpec((SC_CORES, ROW_BLK, 1), lambda m: (0, m, 0)),
            pl.BlockSpec(bias.shape, lambda m: (0,)),
        ],
        out_specs=pl.BlockSpec((ROW_BLK, D_H), lambda m: (m, 0)),
        out_shape=jax.ShapeDtypeStruct((n, D_H), jnp.float32),
    )(acc0, acc1, den, bias)


def _final_body(x_ref, w1_ref, b1_ref, w2_ref, b2_ref, out_ref):
    h = jnp.dot(x_ref[...], w1_ref[...], preferred_element_type=jnp.float32)
    h = jnp.maximum(h + b1_ref[...][None, :], 0.0)
    y = jnp.dot(h, w2_ref[...], preferred_element_type=jnp.float32)
    out_ref[...] = y + b2_ref[...][None, :]


def _final_mlp(x, w1, b1, w2, b2):
    n = x.shape[0]
    grid = n // ROW_BLK
    return pl.pallas_call(
        _final_body,
        grid=(grid,),
        in_specs=[
            pl.BlockSpec((ROW_BLK, D_H), lambda m: (m, 0)),
            pl.BlockSpec(w1.shape, lambda m: (0, 0)),
            pl.BlockSpec(b1.shape, lambda m: (0,)),
            pl.BlockSpec(w2.shape, lambda m: (0, 0)),
            pl.BlockSpec(b2.shape, lambda m: (0,)),
        ],
        out_specs=pl.BlockSpec((ROW_BLK, 1), lambda m: (m, 0)),
        out_shape=jax.ShapeDtypeStruct((n, 1), jnp.float32),
    )(x, w1, b1, w2, b2)


D_HALF = D_H // 2


SCALE_UNROLL = 4


def _sc_conv_body(src_hbm, dst_hbm, asrc_hbm, adst_hbm, hs0_hbm, hs1_hbm,
                  acc0_out, acc1_out, den_out,
                  src_v, dst_v, asrc_v, adst_v, rows0_v, rows1_v, srows_v,
                  exb0_v, exb1_v, exall_v,
                  acc_sh, den_sh, sem_g, sem_s):
    c = lax.axis_index("c")
    s = lax.axis_index("s")
    wid = s * SC_CORES + c
    rows = (rows0_v, rows1_v)
    exb = (exb0_v, exb1_v)
    pltpu.sync_copy(src_hbm.at[wid], src_v)
    pltpu.sync_copy(dst_hbm.at[wid], dst_v)
    pltpu.sync_copy(asrc_hbm, asrc_v)
    pltpu.sync_copy(adst_hbm, adst_v)

    def _zero_rows():
        def _zrow(r, carry):
            for v in range(D_HALF // 16):
                srows_v[r, pl.ds(v * 16, 16)] = jnp.zeros((16,), jnp.float32)
            return carry
        lax.fori_loop(0, CH, _zrow, 0)

    def _zero_acc(include_den):
        _zero_rows()
        if include_den:
            for i in range(CH // 16):
                exb0_v[pl.ds(i * 16, 16)] = jnp.zeros((16,), jnp.float32)
        for k in range(RPT // CH):
            pltpu.sync_copy(srows_v, acc_sh.at[pl.ds(row0 + k * CH, CH)])
            if include_den:
                pltpu.sync_copy(exb0_v, den_sh.at[pl.ds(row0 + k * CH, CH)])

    def _scale_rows(rv, ev):
        @plsc.parallel_loop(0, CH, 1, unroll=SCALE_UNROLL)
        def _scale(r):
            exr = plsc.load_gather(ev, [jnp.full((16,), r, jnp.int32)])
            for g in range(D_HALF // 32):
                v = rv[r, pl.ds(g * 32, 32)]
                lo, hi = plsc.unpack(v, format=plsc.PackFormat.INTERLEAVED,
                                     preferred_element_type=jnp.float32)
                srows_v[r, pl.ds(g * 32, 16)] = lo * exr
                srows_v[r, pl.ds(g * 32 + 16, 16)] = hi * exr

    row0 = s * RPT

    def _run_pass(hs_hbm, first_pass):
        pltpu.async_copy(hs_hbm.at[src_v.at[0]], rows[0], sem_g)

        def _slot(j, p, drain_prev):
            q = 1 - p
            if first_pass:
                base = wid * EPW + j * CH
                for i in range(CH // 16):
                    sv = src_v[j, pl.ds(i * 16, 16)]
                    dv = dst_v[j, pl.ds(i * 16, 16)]
                    a = (plsc.load_gather(asrc_v, [sv])
                         + plsc.load_gather(adst_v, [dv]))
                    a = jnp.maximum(a, 0.2 * a)
                    ex = jnp.exp(a)
                    eid = base + i * 16 + lax.iota(jnp.int32, 16)
                    ex = jnp.where(eid < NUM_EDGES, ex, 0.0)
                    exb[p][pl.ds(i * 16, 16)] = ex
                    exall_v[j, pl.ds(i * 16, 16)] = ex
            else:
                for i in range(CH // 16):
                    exb[p][pl.ds(i * 16, 16)] = exall_v[j, pl.ds(i * 16, 16)]
            pltpu.make_async_copy(hs_hbm.at[src_v.at[j]], rows[p],
                                  sem_g).wait()
            gj = jnp.minimum(j + 1, NCH - 1)
            pltpu.async_copy(hs_hbm.at[src_v.at[gj]], rows[q], sem_g)

            _scale_rows(rows[p], exb[p])
            pltpu.sync_copy(srows_v, acc_sh.at[dst_v.at[j]], add=True)
            if first_pass:
                pltpu.sync_copy(exb[p], den_sh.at[dst_v.at[j]], add=True)

        def _pair(t, carry):
            _slot(2 * t, 0, True)
            _slot(2 * t + 1, 1, True)
            return carry
        lax.fori_loop(0, NCH // 2, _pair, 0)

        pltpu.make_async_copy(hs_hbm.at[src_v.at[NCH - 1]], rows[0],
                              sem_g).wait()

    _zero_acc(True)
    plsc.subcore_barrier()
    _run_pass(hs0_hbm, True)
    plsc.subcore_barrier()

    for k in range(RPT // CH):
        r0 = row0 + k * CH
        pltpu.sync_copy(acc_sh.at[pl.ds(r0, CH)], srows_v)
        pltpu.sync_copy(srows_v, acc0_out.at[c, pl.ds(r0, CH)])
        pltpu.sync_copy(den_sh.at[pl.ds(r0, CH)], exb0_v)
        pltpu.sync_copy(exb0_v, den_out.at[c, pl.ds(r0, CH)])
    _zero_acc(False)
    plsc.subcore_barrier()

    _run_pass(hs1_hbm, False)
    plsc.subcore_barrier()

    for k in range(RPT // CH):
        r0 = row0 + k * CH
        pltpu.sync_copy(acc_sh.at[pl.ds(r0, CH)], srows_v)
        pltpu.sync_copy(srows_v, acc1_out.at[c, pl.ds(r0, CH)])


def _sc_conv(src_r, dst_r, a_src, a_dst, hs0, hs1):
    mesh = plsc.VectorSubcoreMesh(core_axis_name="c", subcore_axis_name="s")
    f = pl.kernel(
        _sc_conv_body,
        out_type=[
            jax.ShapeDtypeStruct((SC_CORES, N_PAD, D_HALF), jnp.float32),
            jax.ShapeDtypeStruct((SC_CORES, N_PAD, D_HALF), jnp.float32),
            jax.ShapeDtypeStruct((SC_CORES, N_PAD), jnp.float32),
        ],
        mesh=mesh,
        scratch_types=[
            pltpu.VMEM((NCH, CH), jnp.int32),
            pltpu.VMEM((NCH, CH), jnp.int32),
            pltpu.VMEM((N_NODES,), jnp.float32),
            pltpu.VMEM((N_NODES,), jnp.float32),
            pltpu.VMEM((CH, D_HALF), jnp.bfloat16),
            pltpu.VMEM((CH, D_HALF), jnp.bfloat16),
            pltpu.VMEM((CH, D_HALF), jnp.float32),
            pltpu.VMEM((CH,), jnp.float32),
            pltpu.VMEM((CH,), jnp.float32),
            pltpu.VMEM((NCH, CH), jnp.float32),
            pltpu.VMEM_SHARED((N_PAD, D_HALF), jnp.float32),
            pltpu.VMEM_SHARED((N_PAD,), jnp.float32),
            pltpu.SemaphoreType.DMA,
            pltpu.SemaphoreType.DMA,
        ],
        compiler_params=pltpu.CompilerParams(needs_layout_passes=False,
                                             use_tc_tiling_on_sc=False),
    )
    return f(src_r, dst_r, a_src, a_dst, hs0, hs1)


def _sparse_middle(hs0, hs1, a_src, a_dst, src_r, dst_r):
    acc0, acc1, den = _sc_conv(src_r, dst_r, a_src.reshape(-1),
                               a_dst.reshape(-1), hs0, hs1)
    return acc0, acc1, den.reshape(SC_CORES, N_PAD, 1)


def _prep_edges(ei):
    ei = ei.astype(jnp.int32)
    src = jnp.pad(ei[0], (0, E_PAD - NUM_EDGES))
    dst = jnp.pad(ei[1], (0, E_PAD - NUM_EDGES))
    return src.reshape(NW, NCH, CH), dst.reshape(NW, NCH, CH)


def kernel(x_inst, x_net, edge_index_i2n, edge_index_n2i, params):
    edges = {
        "i2n": _prep_edges(edge_index_i2n),
        "n2i": _prep_edges(edge_index_n2i),
    }
    for l in range(2):
        p = params["conv"][l]
        outs = []
        for rel, x_src, x_dst in (
            ("i2n", x_inst, x_net),
            ("n2i", x_net, x_inst),
        ):
            pp = p[rel]
            src_r, dst_r = edges[rel]
            hs0, hs1, a_src, a_dst = _gat_pre(x_src, x_dst, pp)
            acc0, acc1, den = _sparse_middle(hs0, hs1, a_src, a_dst,
                                             src_r, dst_r)
            outs.append(_gat_post(acc0, acc1, den, pp["bias"], x_dst.shape[0]))
        x_net, x_inst = outs
    x = _final_mlp(x_net, params["lin1_W"], params["lin1_b"],
                   params["lin2_W"], params["lin2_b"])
    return x


# speedup vs baseline: 2.8243x; 1.4773x over previous
import functools

import jax
import jax.numpy as jnp
import numpy as np
from jax import lax
from jax.experimental import pallas as pl
from jax.experimental.pallas import tpu as pltpu
from jax.experimental.pallas import tpu_sc as plsc

_PERM32 = np.empty(32, np.int32)
_PERM32[0::2] = np.arange(16)
_PERM32[1::2] = np.arange(16, 32)


def _bf16_col_perm(width):
    return np.concatenate([g * 32 + _PERM32 for g in range(width // 32)])

N_NODES = 10000
NUM_EDGES = 320000
D_IN = 128
D_H = 128
D_OUT = 64
ROW_BLK = 2000

SC_CORES = 2
SC_TILES = 16
NW = SC_CORES * SC_TILES
CH = 128
EPW = 10240
NCH = EPW // CH
E_PAD = NW * EPW
N_PAD = 10240
RPT = N_PAD // SC_TILES


def _pre_body(xs_ref, xd_ref, w0_ref, w1_ref, ws_ref, wd_ref,
              avs_ref, avd_ref, hs0_ref, hs1_ref, asrc_ref, adst_ref):
    xs = xs_ref[...]
    hs0_ref[...] = jnp.dot(xs, w0_ref[...],
                           preferred_element_type=jnp.float32
                           ).astype(jnp.bfloat16)
    hs1_ref[...] = jnp.dot(xs, w1_ref[...],
                           preferred_element_type=jnp.float32
                           ).astype(jnp.bfloat16)
    wsa = jnp.dot(ws_ref[...], avs_ref[...][:, None],
                  preferred_element_type=jnp.float32)
    asrc_ref[...] = jnp.dot(xs, wsa, preferred_element_type=jnp.float32)
    wda = jnp.dot(wd_ref[...], avd_ref[...][:, None],
                  preferred_element_type=jnp.float32)
    adst_ref[...] = jnp.dot(xd_ref[...], wda, preferred_element_type=jnp.float32)


def _gat_pre(x_src, x_dst, p):
    n = x_src.shape[0]
    grid = n // ROW_BLK
    perm = _bf16_col_perm(D_H // 2)
    w0 = p["W_src"][:, :D_H // 2][:, perm]
    w1 = p["W_src"][:, D_H // 2:][:, perm]
    return pl.pallas_call(
        _pre_body,
        grid=(grid,),
        in_specs=[
            pl.BlockSpec((ROW_BLK, x_src.shape[1]), lambda m: (m, 0)),
            pl.BlockSpec((ROW_BLK, x_dst.shape[1]), lambda m: (m, 0)),
            pl.BlockSpec(w0.shape, lambda m: (0, 0)),
            pl.BlockSpec(w1.shape, lambda m: (0, 0)),
            pl.BlockSpec(p["W_src"].shape, lambda m: (0, 0)),
            pl.BlockSpec(p["W_dst"].shape, lambda m: (0, 0)),
            pl.BlockSpec(p["att_src"].shape, lambda m: (0,)),
            pl.BlockSpec(p["att_dst"].shape, lambda m: (0,)),
        ],
        out_specs=[
            pl.BlockSpec((ROW_BLK, D_H // 2), lambda m: (m, 0)),
            pl.BlockSpec((ROW_BLK, D_H // 2), lambda m: (m, 0)),
            pl.BlockSpec((ROW_BLK, 1), lambda m: (m, 0)),
            pl.BlockSpec((ROW_BLK, 1), lambda m: (m, 0)),
        ],
        out_shape=[
            jax.ShapeDtypeStruct((n, D_H // 2), jnp.bfloat16),
            jax.ShapeDtypeStruct((n, D_H // 2), jnp.bfloat16),
            jax.ShapeDtypeStruct((n, 1), jnp.float32),
            jax.ShapeDtypeStruct((n, 1), jnp.float32),
        ],
    )(x_src, x_dst, w0, w1, p["W_src"], p["W_dst"],
      p["att_src"], p["att_dst"])


def _post_body(acc0_ref, acc1_ref, den_ref, bias_ref, out_ref):
    acc0 = acc0_ref[...]
    acc1 = acc1_ref[...]
    den = den_ref[...]
    acc_t = jnp.concatenate([acc0[0] + acc0[1], acc1[0] + acc1[1]], axis=-1)
    den_t = den[0] + den[1]
    out = acc_t / jnp.maximum(den_t, 1e-16)
    out_ref[...] = jnp.maximum(out + bias_ref[...][None, :], 0.0)


def _gat_post(acc0, acc1, den, bias, n):
    grid = n // ROW_BLK
    return pl.pallas_call(
        _post_body,
        grid=(grid,),
        in_specs=[
            pl.BlockSpec((SC_CORES, ROW_BLK, D_H // 2), lambda m: (0, m, 0)),
            pl.BlockSpec((SC_CORES, ROW_BLK, D_H // 2), lambda m: (0, m, 0)),
            pl.BlockSpec((SC_CORES, ROW_BLK, 1), lambda m: (0, m, 0)),
            pl.BlockSpec(bias.shape, lambda m: (0,)),
        ],
        out_specs=pl.BlockSpec((ROW_BLK, D_H), lambda m: (m, 0)),
        out_shape=jax.ShapeDtypeStruct((n, D_H), jnp.float32),
    )(acc0, acc1, den, bias)


def _final_body(x_ref, w1_ref, b1_ref, w2_ref, b2_ref, out_ref):
    h = jnp.dot(x_ref[...], w1_ref[...], preferred_element_type=jnp.float32)
    h = jnp.maximum(h + b1_ref[...][None, :], 0.0)
    y = jnp.dot(h, w2_ref[...], preferred_element_type=jnp.float32)
    out_ref[...] = y + b2_ref[...][None, :]


def _final_mlp(x, w1, b1, w2, b2):
    n = x.shape[0]
    grid = n // ROW_BLK
    return pl.pallas_call(
        _final_body,
        grid=(grid,),
        in_specs=[
            pl.BlockSpec((ROW_BLK, D_H), lambda m: (m, 0)),
            pl.BlockSpec(w1.shape, lambda m: (0, 0)),
            pl.BlockSpec(b1.shape, lambda m: (0,)),
            pl.BlockSpec(w2.shape, lambda m: (0, 0)),
            pl.BlockSpec(b2.shape, lambda m: (0,)),
        ],
        out_specs=pl.BlockSpec((ROW_BLK, 1), lambda m: (m, 0)),
        out_shape=jax.ShapeDtypeStruct((n, 1), jnp.float32),
    )(x, w1, b1, w2, b2)


D_HALF = D_H // 2


SCALE_UNROLL = 4


def _sc_conv_body(src_hbm, dst_hbm, asrc_hbm, adst_hbm, hs0_hbm, hs1_hbm,
                  acc0_out, acc1_out, den_out,
                  src_v, dst_v, asrc_v, adst_v, rows0_v, rows1_v, srows_v,
                  exb0_v, exb1_v, exall_v,
                  acc_sh, den_sh, tbl_sh, sem_g, sem_s):
    c = lax.axis_index("c")
    s = lax.axis_index("s")
    wid = s * SC_CORES + c
    rows = (rows0_v, rows1_v)
    exb = (exb0_v, exb1_v)
    pltpu.sync_copy(src_hbm.at[wid], src_v)
    pltpu.sync_copy(dst_hbm.at[wid], dst_v)
    pltpu.sync_copy(asrc_hbm, asrc_v)
    pltpu.sync_copy(adst_hbm, adst_v)

    def _zero_rows():
        def _zrow(r, carry):
            for v in range(D_HALF // 16):
                srows_v[r, pl.ds(v * 16, 16)] = jnp.zeros((16,), jnp.float32)
            return carry
        lax.fori_loop(0, CH, _zrow, 0)

    def _zero_acc(include_den):
        _zero_rows()
        if include_den:
            for i in range(CH // 16):
                exb0_v[pl.ds(i * 16, 16)] = jnp.zeros((16,), jnp.float32)
        for k in range(RPT // CH):
            pltpu.sync_copy(srows_v, acc_sh.at[pl.ds(row0 + k * CH, CH)])
            if include_den:
                pltpu.sync_copy(exb0_v, den_sh.at[pl.ds(row0 + k * CH, CH)])

    def _scale_rows(rv, ev):
        @plsc.parallel_loop(0, CH, 1, unroll=SCALE_UNROLL)
        def _scale(r):
            exr = plsc.load_gather(ev, [jnp.full((16,), r, jnp.int32)])
            for g in range(D_HALF // 32):
                v = rv[r, pl.ds(g * 32, 32)]
                lo, hi = plsc.unpack(v, format=plsc.PackFormat.INTERLEAVED,
                                     preferred_element_type=jnp.float32)
                srows_v[r, pl.ds(g * 32, 16)] = lo * exr
                srows_v[r, pl.ds(g * 32 + 16, 16)] = hi * exr

    row0 = s * RPT

    def _stage_table(hs_hbm):
        t0 = s * (N_NODES // SC_TILES)
        pltpu.sync_copy(hs_hbm.at[pl.ds(t0, N_NODES // SC_TILES)],
                        tbl_sh.at[pl.ds(t0, N_NODES // SC_TILES)])

    def _run_pass(first_pass):
        pltpu.async_copy(tbl_sh.at[src_v.at[0]], rows[0], sem_g)

        def _slot(j, p, drain_prev):
            q = 1 - p
            if first_pass:
                base = wid * EPW + j * CH
                for i in range(CH // 16):
                    sv = src_v[j, pl.ds(i * 16, 16)]
                    dv = dst_v[j, pl.ds(i * 16, 16)]
                    a = (plsc.load_gather(asrc_v, [sv])
                         + plsc.load_gather(adst_v, [dv]))
                    a = jnp.maximum(a, 0.2 * a)
                    ex = jnp.exp(a)
                    eid = base + i * 16 + lax.iota(jnp.int32, 16)
                    ex = jnp.where(eid < NUM_EDGES, ex, 0.0)
                    exb[p][pl.ds(i * 16, 16)] = ex
                    exall_v[j, pl.ds(i * 16, 16)] = ex
            else:
                for i in range(CH // 16):
                    exb[p][pl.ds(i * 16, 16)] = exall_v[j, pl.ds(i * 16, 16)]
            pltpu.make_async_copy(tbl_sh.at[src_v.at[j]], rows[p],
                                  sem_g).wait()
            gj = jnp.minimum(j + 1, NCH - 1)
            pltpu.async_copy(tbl_sh.at[src_v.at[gj]], rows[q], sem_g)

            _scale_rows(rows[p], exb[p])
            pltpu.sync_copy(srows_v, acc_sh.at[dst_v.at[j]], add=True)
            if first_pass:
                pltpu.sync_copy(exb[p], den_sh.at[dst_v.at[j]], add=True)

        def _pair(t, carry):
            _slot(2 * t, 0, True)
            _slot(2 * t + 1, 1, True)
            return carry
        lax.fori_loop(0, NCH // 2, _pair, 0)

        pltpu.make_async_copy(tbl_sh.at[src_v.at[NCH - 1]], rows[0],
                              sem_g).wait()

    _stage_table(hs0_hbm)
    _zero_acc(True)
    plsc.subcore_barrier()
    _run_pass(True)
    plsc.subcore_barrier()

    for k in range(RPT // CH):
        r0 = row0 + k * CH
        pltpu.sync_copy(acc_sh.at[pl.ds(r0, CH)], srows_v)
        pltpu.sync_copy(srows_v, acc0_out.at[c, pl.ds(r0, CH)])
        pltpu.sync_copy(den_sh.at[pl.ds(r0, CH)], exb0_v)
        pltpu.sync_copy(exb0_v, den_out.at[c, pl.ds(r0, CH)])
    _stage_table(hs1_hbm)
    _zero_acc(False)
    plsc.subcore_barrier()

    _run_pass(False)
    plsc.subcore_barrier()

    for k in range(RPT // CH):
        r0 = row0 + k * CH
        pltpu.sync_copy(acc_sh.at[pl.ds(r0, CH)], srows_v)
        pltpu.sync_copy(srows_v, acc1_out.at[c, pl.ds(r0, CH)])


def _sc_conv(src_r, dst_r, a_src, a_dst, hs0, hs1):
    mesh = plsc.VectorSubcoreMesh(core_axis_name="c", subcore_axis_name="s")
    f = pl.kernel(
        _sc_conv_body,
        out_type=[
            jax.ShapeDtypeStruct((SC_CORES, N_PAD, D_HALF), jnp.float32),
            jax.ShapeDtypeStruct((SC_CORES, N_PAD, D_HALF), jnp.float32),
            jax.ShapeDtypeStruct((SC_CORES, N_PAD), jnp.float32),
        ],
        mesh=mesh,
        scratch_types=[
            pltpu.VMEM((NCH, CH), jnp.int32),
            pltpu.VMEM((NCH, CH), jnp.int32),
            pltpu.VMEM((N_NODES,), jnp.float32),
            pltpu.VMEM((N_NODES,), jnp.float32),
            pltpu.VMEM((CH, D_HALF), jnp.bfloat16),
            pltpu.VMEM((CH, D_HALF), jnp.bfloat16),
            pltpu.VMEM((CH, D_HALF), jnp.float32),
            pltpu.VMEM((CH,), jnp.float32),
            pltpu.VMEM((CH,), jnp.float32),
            pltpu.VMEM((NCH, CH), jnp.float32),
            pltpu.VMEM_SHARED((N_PAD, D_HALF), jnp.float32),
            pltpu.VMEM_SHARED((N_PAD,), jnp.float32),
            pltpu.VMEM_SHARED((N_NODES, D_HALF), jnp.bfloat16),
            pltpu.SemaphoreType.DMA,
            pltpu.SemaphoreType.DMA,
        ],
        compiler_params=pltpu.CompilerParams(needs_layout_passes=False,
                                             use_tc_tiling_on_sc=False),
    )
    return f(src_r, dst_r, a_src, a_dst, hs0, hs1)


def _sparse_middle(hs0, hs1, a_src, a_dst, src_r, dst_r):
    acc0, acc1, den = _sc_conv(src_r, dst_r, a_src.reshape(-1),
                               a_dst.reshape(-1), hs0, hs1)
    return acc0, acc1, den.reshape(SC_CORES, N_PAD, 1)


def _prep_edges(ei):
    ei = ei.astype(jnp.int32)
    src = jnp.pad(ei[0], (0, E_PAD - NUM_EDGES))
    dst = jnp.pad(ei[1], (0, E_PAD - NUM_EDGES))
    return src.reshape(NW, NCH, CH), dst.reshape(NW, NCH, CH)


def kernel(x_inst, x_net, edge_index_i2n, edge_index_n2i, params):
    edges = {
        "i2n": _prep_edges(edge_index_i2n),
        "n2i": _prep_edges(edge_index_n2i),
    }
    for l in range(2):
        p = params["conv"][l]
        outs = []
        for rel, x_src, x_dst in (
            ("i2n", x_inst, x_net),
            ("n2i", x_net, x_inst),
        ):
            pp = p[rel]
            src_r, dst_r = edges[rel]
            hs0, hs1, a_src, a_dst = _gat_pre(x_src, x_dst, pp)
            acc0, acc1, den = _sparse_middle(hs0, hs1, a_src, a_dst,
                                             src_r, dst_r)
            outs.append(_gat_post(acc0, acc1, den, pp["bias"], x_dst.shape[0]))
        x_net, x_inst = outs
    x = _final_mlp(x_net, params["lin1_W"], params["lin1_b"],
                   params["lin2_W"], params["lin2_b"])
    return x
